# baseline (speedup 1.0000x reference)
import functools

import jax
import jax.numpy as jnp
from jax import lax
from jax.experimental import pallas as pl
from jax.experimental.pallas import tpu as pltpu
from jax.experimental.pallas import tpu_sc as plsc

N = 10000
E = 320000
NROW = 10240
G = 64
NC, NS = 2, 16
NW = NC * NS

_EMB_CH = 80
_EMB_PER_TILE = NROW // NW


def _sc_embed_body(ports_hbm, ptab2_hbm, epp_hbm, pidx, qidx, prow_v, sem):
    wid = lax.axis_index("s") * NC + lax.axis_index("c")
    base0 = wid * _EMB_PER_TILE
    for k in range(_EMB_PER_TILE // _EMB_CH):
        base = base0 + k * _EMB_CH
        pltpu.sync_copy(ports_hbm.at[pl.ds(base, _EMB_CH)], pidx)
        for g in range(_EMB_CH // 16):
            qidx[pl.ds(16 * g, 16)] = lax.shift_right_logical(
                pidx[pl.ds(16 * g, 16)], 3)
        pltpu.async_copy(ptab2_hbm.at[qidx], prow_v, sem).wait()
        pltpu.sync_copy(prow_v, epp_hbm.at[pl.ds(base, _EMB_CH)])


@functools.cache
def _sc_embed():
    return pl.kernel(
        _sc_embed_body,
        out_type=jax.ShapeDtypeStruct((NROW, 128), jnp.float32),
        mesh=plsc.VectorSubcoreMesh(core_axis_name="c", subcore_axis_name="s",
                                    num_cores=NC, num_subcores=NS),
        scratch_types=[pltpu.VMEM((_EMB_CH,), jnp.int32),
                       pltpu.VMEM((_EMB_CH,), jnp.int32),
                       pltpu.VMEM((_EMB_CH, 128), jnp.float32),
                       pltpu.SemaphoreType.DMA],
    )

_ECH = 128
_EPAD = 327680
_STG = 40
_SLOW_CID = 1
_SLOW_BLK = 1
_FAST_BLK = 3
_RPT = NROW // NS


def _sc_agg_body(m_hbm, src_hbm, dst_hbm, zro_hbm, o0_hbm, o1_hbm,
                 acc, srci, dsti, rows_a, rows_b,
                 zsem, isem, gsem_a, gsem_b, ssem_a, ssem_b):
    cid = lax.axis_index("c")
    sid = lax.axis_index("s")
    rbase = sid * _RPT

    def g_start(c, buf, sem):
        pltpu.async_copy(m_hbm.at[srci.at[c]], buf, sem)

    def g_wait(c, buf, sem):
        pltpu.make_async_copy(m_hbm.at[srci.at[c]], buf, sem).wait()

    def s_start(c, buf, sem):
        pltpu.async_copy(buf, acc.at[dsti.at[c]], sem, add=True)

    def s_wait(c, buf, sem):
        pltpu.make_async_copy(buf, acc.at[dsti.at[c]], sem).wait()

    def stage_idx(trow):
        pltpu.async_copy(src_hbm.at[pl.ds(trow, _STG)], srci, isem)
        pltpu.async_copy(dst_hbm.at[pl.ds(trow, _STG)], dsti, isem)
        pltpu.make_async_copy(src_hbm.at[pl.ds(trow, _STG)], srci, isem).wait()
        pltpu.make_async_copy(dst_hbm.at[pl.ds(trow, _STG)], dsti, isem).wait()

    def pair(j, carry):
        a = 2 * j
        b = a + 1
        g_wait(a, rows_a, gsem_a)
        s_start(a, rows_a, ssem_a)
        g_wait(b, rows_b, gsem_b)
        s_start(b, rows_b, ssem_b)
        s_wait(a, rows_a, ssem_a)
        g_start(a + 2, rows_a, gsem_a)
        s_wait(b, rows_b, ssem_b)
        g_start(b + 2, rows_b, gsem_b)
        return carry

    def run_block(trow):
        stage_idx(trow)
        g_start(0, rows_a, gsem_a)
        g_start(1, rows_b, gsem_b)
        lax.fori_loop(0, _STG // 2 - 1, pair, 0)
        a = _STG - 2
        g_wait(a, rows_a, gsem_a)
        s_start(a, rows_a, ssem_a)
        g_wait(a + 1, rows_b, gsem_b)
        s_start(a + 1, rows_b, ssem_b)
        s_wait(a, rows_a, ssem_a)
        s_wait(a + 1, rows_b, ssem_b)

    pltpu.async_copy(zro_hbm, acc.at[pl.ds(rbase, _RPT)], zsem)
    pltpu.make_async_copy(zro_hbm, acc.at[pl.ds(rbase, _RPT)], zsem).wait()
    plsc.subcore_barrier()

    @pl.when(cid == _SLOW_CID)
    def _():
        for st in range(_SLOW_BLK):
            run_block((sid * _SLOW_BLK + st) * _STG)

    @pl.when(cid != _SLOW_CID)
    def _():
        base = NS * _SLOW_BLK * _STG
        for st in range(_FAST_BLK):
            run_block(base + (sid * _FAST_BLK + st) * _STG)

    plsc.subcore_barrier()

    @pl.when(cid == 0)
    def _():
        pltpu.sync_copy(acc.at[pl.ds(rbase, _RPT)], o0_hbm.at[pl.ds(rbase, _RPT)])

    @pl.when(cid == 1)
    def _():
        pltpu.sync_copy(acc.at[pl.ds(rbase, _RPT)], o1_hbm.at[pl.ds(rbase, _RPT)])


@functools.cache
def _sc_agg():
    return pl.kernel(
        _sc_agg_body,
        out_type=(jax.ShapeDtypeStruct((NROW, 128), jnp.float32),
                  jax.ShapeDtypeStruct((NROW, 128), jnp.float32)),
        mesh=plsc.VectorSubcoreMesh(core_axis_name="c", subcore_axis_name="s",
                                    num_cores=NC, num_subcores=NS),
        scratch_types=[pltpu.VMEM_SHARED((NROW, 128), jnp.float32),
                       pltpu.VMEM((_STG, _ECH), jnp.int32),
                       pltpu.VMEM((_STG, _ECH), jnp.int32),
                       pltpu.VMEM((_ECH, 128), jnp.float32),
                       pltpu.VMEM((_ECH, 128), jnp.float32),
                       pltpu.SemaphoreType.DMA,
                       pltpu.SemaphoreType.DMA,
                       pltpu.SemaphoreType.DMA,
                       pltpu.SemaphoreType.DMA,
                       pltpu.SemaphoreType.DMA,
                       pltpu.SemaphoreType.DMA],
    )

_BR = 512
_GRID = NROW // _BR


def _dot(a, b):
    return jnp.dot(a, b, preferred_element_type=jnp.float32)


def _tc_in_body(x_ref, epp_ref, pbc_ref, fbc_ref, ttab_ref,
                wxr_ref, wpr_ref, wtr_ref, wxo_ref, wpo_ref, wto_ref,
                m_ref, r_ref):
    xa = x_ref[...]
    lane_grp = lax.shift_right_logical(
        lax.broadcasted_iota(jnp.int32, (_BR, 128), 1), 4)
    rem = pbc_ref[...] & 7
    pm = jnp.where(lane_grp == rem, epp_ref[...], 0.0)
    oh = jnp.where(
        fbc_ref[...] == lax.broadcasted_iota(jnp.int32, (_BR, 256), 1),
        1.0, 0.0)
    eta = _dot(oh, ttab_ref[...])
    m_ref[...] = (_dot(xa, wxr_ref[...]) + _dot(pm, wpr_ref[...])
                  + _dot(eta, wtr_ref[...]))
    r_ref[...] = (_dot(xa, wxo_ref[...]) + _dot(pm, wpo_ref[...])
                  + _dot(eta, wto_ref[...]))


_row_spec = pl.BlockSpec((_BR, 128), lambda i: (i, 0))
_emb_spec = pl.BlockSpec((_BR, 16), lambda i: (i, 0))
_w128_spec = pl.BlockSpec((128, 128), lambda i: (0, 0))
_w16_spec = pl.BlockSpec((16, 128), lambda i: (0, 0))
_b_spec = pl.BlockSpec((1, 128), lambda i: (0, 0))

_fbc_spec = pl.BlockSpec((_BR, 256), lambda i: (i, 0))
_ttab_spec = pl.BlockSpec((256, 16), lambda i: (0, 0))

_tc_in = pl.pallas_call(
    _tc_in_body,
    grid=(_GRID,),
    in_specs=[_row_spec, _row_spec, _row_spec, _fbc_spec, _ttab_spec,
              _w128_spec, _w128_spec, _w16_spec,
              _w128_spec, _w128_spec, _w16_spec],
    out_specs=(_row_spec, _row_spec),
    out_shape=(jax.ShapeDtypeStruct((NROW, 128), jnp.float32),
               jax.ShapeDtypeStruct((NROW, 128), jnp.float32)),
)


def _tc_mid_body(a0_ref, a1_ref, r_ref, b_ref, wr_ref, wo_ref, m_ref, r2_ref):
    h = jnp.maximum(a0_ref[...] + a1_ref[...] + r_ref[...] + b_ref[...], 0.0)
    m_ref[...] = _dot(h, wr_ref[...])
    r2_ref[...] = _dot(h, wo_ref[...])


_tc_mid = pl.pallas_call(
    _tc_mid_body,
    grid=(_GRID,),
    in_specs=[_row_spec, _row_spec, _row_spec, _b_spec, _w128_spec, _w128_spec],
    out_specs=(_row_spec, _row_spec),
    out_shape=(jax.ShapeDtypeStruct((NROW, 128), jnp.float32),
               jax.ShapeDtypeStruct((NROW, 128), jnp.float32)),
)


def _tc_fin_body(a0_ref, a1_ref, r_ref, b_ref, bat_ref,
                 wf1_ref, bf1_ref, wf2_ref, bf2_ref, wf3_ref, bf3_ref,
                 out_ref, pooled):
    i = pl.program_id(0)

    @pl.when(i == 0)
    def _():
        pooled[...] = jnp.full((G, 128), -jnp.inf, jnp.float32)

    h = jnp.maximum(a0_ref[...] + a1_ref[...] + r_ref[...] + b_ref[...], 0.0)
    bat = bat_ref[...]
    gmin = jnp.min(bat)
    gmax = jnp.max(bat)

    def gbody(g, carry):
        sel = jnp.where(bat == g, h, -jnp.inf)
        loc = jnp.max(sel, axis=0, keepdims=True)
        gc = jnp.minimum(g, G - 1)

        @pl.when(g < G)
        def _():
            pooled[pl.ds(gc, 1), :] = jnp.maximum(pooled[pl.ds(gc, 1), :], loc)

        return carry

    lax.fori_loop(gmin, gmax + 1, gbody, 0)

    @pl.when(i == _GRID - 1)
    def _():
        p = pooled[...]
        z = jnp.maximum(_dot(p, wf1_ref[...]) + bf1_ref[...], 0.0)
        z = jnp.maximum(_dot(z, wf2_ref[...]) + bf2_ref[...], 0.0)
        out_ref[...] = _dot(z, wf3_ref[...]) + bf3_ref[...]


_tc_fin = pl.pallas_call(
    _tc_fin_body,
    grid=(_GRID,),
    in_specs=[_row_spec, _row_spec, _row_spec, _b_spec, _row_spec,
              _w128_spec, _b_spec, _w128_spec, _b_spec, _w128_spec, _b_spec],
    out_specs=pl.BlockSpec((G, 128), lambda i: (0, 0)),
    out_shape=jax.ShapeDtypeStruct((G, 128), jnp.float32),
    scratch_shapes=[pltpu.VMEM((G, 128), jnp.float32)],
)


def kernel(x, dst_ports, tcp_flags, edge_index, batch, dst_table, tcp_table,
           Wr0, br0, Wo0, Wr1, br1, Wo1, Wr2, br2, Wo2,
           Wf1, bf1, Wf2, bf2, Wf3, bf3):
    ports = jnp.pad(dst_ports.astype(jnp.int32), (0, NROW - N))
    flags = jnp.pad(tcp_flags.astype(jnp.int32), (0, NROW - N))
    ptab2 = dst_table.reshape(8192, 128)
    epp = _sc_embed()(ports, ptab2)

    src2d = jnp.pad(edge_index[0].astype(jnp.int32),
                    (0, _EPAD - E)).reshape(_EPAD // _ECH, _ECH)
    dst_pad = N + (jnp.arange(_EPAD - E, dtype=jnp.int32) % (NROW - N))
    dst2d = jnp.concatenate(
        [edge_index[1].astype(jnp.int32), dst_pad]).reshape(_EPAD // _ECH, _ECH)

    zrows = jnp.zeros((_RPT, 128), jnp.float32)
    ports_bc = jnp.broadcast_to(ports[:, None], (NROW, 128))
    flags_bc = jnp.broadcast_to(flags[:, None], (NROW, 256))
    ttab16 = jnp.pad(tcp_table, ((0, 0), (0, 14)))
    m, r = _tc_in(x, epp, ports_bc, flags_bc, ttab16,
                  Wr0[:128], jnp.tile(Wr0[128:144], (8, 1)),
                  jnp.pad(Wr0[144:146], ((0, 14), (0, 0))),
                  Wo0[:128], jnp.tile(Wo0[128:144], (8, 1)),
                  jnp.pad(Wo0[144:146], ((0, 14), (0, 0))))
    a0, a1 = _sc_agg()(m, src2d, dst2d, zrows)
    m, r = _tc_mid(a0, a1, r, br0.reshape(1, 128), Wr1, Wo1)
    a0, a1 = _sc_agg()(m, src2d, dst2d, zrows)
    m, r = _tc_mid(a0, a1, r, br1.reshape(1, 128), Wr2, Wo2)
    a0, a1 = _sc_agg()(m, src2d, dst2d, zrows)

    batp = jnp.broadcast_to(
        jnp.pad(batch.astype(jnp.int32), (0, NROW - N), constant_values=G)[:, None],
        (NROW, 128))
    outf = _tc_fin(a0, a1, r, br2.reshape(1, 128), batp,
                   Wf1, bf1.reshape(1, 128), Wf2, bf2.reshape(1, 128),
                   jnp.pad(Wf3, ((0, 0), (0, 118))),
                   jnp.pad(bf3, (0, 118)).reshape(1, 128))
    return outf[:, :10]


# speedup vs baseline: 2.8950x; 2.8950x over previous
import functools

import jax
import jax.numpy as jnp
from jax import lax
from jax.experimental import pallas as pl
from jax.experimental.pallas import tpu as pltpu
from jax.experimental.pallas import tpu_sc as plsc

N = 10000
E = 320000
NROW = 10240
G = 64
NC, NS = 2, 16
NW = NC * NS

_EMB_CH = 80
_EMB_PER_TILE = NROW // NW


def _sc_embed_body(ports_hbm, ptab2_hbm, epp_hbm, pidx, qidx, prow_v, sem):
    wid = lax.axis_index("s") * NC + lax.axis_index("c")
    base0 = wid * _EMB_PER_TILE
    for k in range(_EMB_PER_TILE // _EMB_CH):
        base = base0 + k * _EMB_CH
        pltpu.sync_copy(ports_hbm.at[pl.ds(base, _EMB_CH)], pidx)
        for g in range(_EMB_CH // 16):
            qidx[pl.ds(16 * g, 16)] = lax.shift_right_logical(
                pidx[pl.ds(16 * g, 16)], 3)
        pltpu.async_copy(ptab2_hbm.at[qidx], prow_v, sem).wait()
        pltpu.sync_copy(prow_v, epp_hbm.at[pl.ds(base, _EMB_CH)])


@functools.cache
def _sc_embed():
    return pl.kernel(
        _sc_embed_body,
        out_type=jax.ShapeDtypeStruct((NROW, 128), jnp.float32),
        mesh=plsc.VectorSubcoreMesh(core_axis_name="c", subcore_axis_name="s",
                                    num_cores=NC, num_subcores=NS),
        scratch_types=[pltpu.VMEM((_EMB_CH,), jnp.int32),
                       pltpu.VMEM((_EMB_CH,), jnp.int32),
                       pltpu.VMEM((_EMB_CH, 128), jnp.float32),
                       pltpu.SemaphoreType.DMA],
    )

_ECH = 128
_EPAD = 327680
_STG = 40
_SLOW_CID = 1
_SLOW_BLK = 2
_FAST_BLK = 2
_RPT = NROW // NS


def _sc_agg_body(m_hbm, src_hbm, dst_hbm, zro_hbm, o0_hbm, o1_hbm,
                 acc, srci, dsti, rows_a, rows_b,
                 zsem, isem, gsem_a, gsem_b, ssem_a, ssem_b):
    cid = lax.axis_index("c")
    sid = lax.axis_index("s")
    rbase = sid * _RPT

    def g_start(c, buf, sem):
        pltpu.async_copy(m_hbm.at[srci.at[c]], buf, sem)

    def g_wait(c, buf, sem):
        pltpu.make_async_copy(m_hbm.at[srci.at[c]], buf, sem).wait()

    def s_start(c, buf, sem):
        pltpu.async_copy(buf, acc.at[dsti.at[c]], sem, add=True)

    def s_wait(c, buf, sem):
        pltpu.make_async_copy(buf, acc.at[dsti.at[c]], sem).wait()

    def stage_idx(trow):
        pltpu.async_copy(src_hbm.at[pl.ds(trow, _STG)], srci, isem)
        pltpu.async_copy(dst_hbm.at[pl.ds(trow, _STG)], dsti, isem)
        pltpu.make_async_copy(src_hbm.at[pl.ds(trow, _STG)], srci, isem).wait()
        pltpu.make_async_copy(dst_hbm.at[pl.ds(trow, _STG)], dsti, isem).wait()

    def pair(j, carry):
        a = 2 * j
        b = a + 1
        g_wait(a, rows_a, gsem_a)
        s_start(a, rows_a, ssem_a)
        g_wait(b, rows_b, gsem_b)
        s_start(b, rows_b, ssem_b)
        s_wait(a, rows_a, ssem_a)
        g_start(a + 2, rows_a, gsem_a)
        s_wait(b, rows_b, ssem_b)
        g_start(b + 2, rows_b, gsem_b)
        return carry

    def run_block(trow):
        stage_idx(trow)
        g_start(0, rows_a, gsem_a)
        g_start(1, rows_b, gsem_b)
        lax.fori_loop(0, _STG // 2 - 1, pair, 0)
        a = _STG - 2
        g_wait(a, rows_a, gsem_a)
        s_start(a, rows_a, ssem_a)
        g_wait(a + 1, rows_b, gsem_b)
        s_start(a + 1, rows_b, ssem_b)
        s_wait(a, rows_a, ssem_a)
        s_wait(a + 1, rows_b, ssem_b)

    pltpu.async_copy(zro_hbm, acc.at[pl.ds(rbase, _RPT)], zsem)
    pltpu.make_async_copy(zro_hbm, acc.at[pl.ds(rbase, _RPT)], zsem).wait()
    plsc.subcore_barrier()

    @pl.when(cid == _SLOW_CID)
    def _():
        for st in range(_SLOW_BLK):
            run_block((sid * _SLOW_BLK + st) * _STG)

    @pl.when(cid != _SLOW_CID)
    def _():
        base = NS * _SLOW_BLK * _STG
        for st in range(_FAST_BLK):
            run_block(base + (sid * _FAST_BLK + st) * _STG)

    plsc.subcore_barrier()

    @pl.when(cid == 0)
    def _():
        pltpu.sync_copy(acc.at[pl.ds(rbase, _RPT)], o0_hbm.at[pl.ds(rbase, _RPT)])

    @pl.when(cid == 1)
    def _():
        pltpu.sync_copy(acc.at[pl.ds(rbase, _RPT)], o1_hbm.at[pl.ds(rbase, _RPT)])


@functools.cache
def _sc_agg():
    return pl.kernel(
        _sc_agg_body,
        out_type=(jax.ShapeDtypeStruct((NROW, 128), jnp.float32),
                  jax.ShapeDtypeStruct((NROW, 128), jnp.float32)),
        mesh=plsc.VectorSubcoreMesh(core_axis_name="c", subcore_axis_name="s",
                                    num_cores=NC, num_subcores=NS),
        scratch_types=[pltpu.VMEM_SHARED((NROW, 128), jnp.float32),
                       pltpu.VMEM((_STG, _ECH), jnp.int32),
                       pltpu.VMEM((_STG, _ECH), jnp.int32),
                       pltpu.VMEM((_ECH, 128), jnp.float32),
                       pltpu.VMEM((_ECH, 128), jnp.float32),
                       pltpu.SemaphoreType.DMA,
                       pltpu.SemaphoreType.DMA,
                       pltpu.SemaphoreType.DMA,
                       pltpu.SemaphoreType.DMA,
                       pltpu.SemaphoreType.DMA,
                       pltpu.SemaphoreType.DMA],
    )

_BR = 512
_GRID = NROW // _BR


def _dot(a, b):
    return jnp.dot(a, b, preferred_element_type=jnp.float32)


def _tc_in_body(x_ref, epp_ref, pbc_ref, fbc_ref, ttab_ref,
                wxr_ref, wpr_ref, wtr_ref, wxo_ref, wpo_ref, wto_ref,
                m_ref, r_ref):
    xa = x_ref[...]
    lane_grp = lax.shift_right_logical(
        lax.broadcasted_iota(jnp.int32, (_BR, 128), 1), 4)
    rem = pbc_ref[...] & 7
    pm = jnp.where(lane_grp == rem, epp_ref[...], 0.0)
    oh = jnp.where(
        fbc_ref[...] == lax.broadcasted_iota(jnp.int32, (_BR, 256), 1),
        1.0, 0.0)
    eta = _dot(oh, ttab_ref[...])
    m_ref[...] = (_dot(xa, wxr_ref[...]) + _dot(pm, wpr_ref[...])
                  + _dot(eta, wtr_ref[...]))
    r_ref[...] = (_dot(xa, wxo_ref[...]) + _dot(pm, wpo_ref[...])
                  + _dot(eta, wto_ref[...]))


_row_spec = pl.BlockSpec((_BR, 128), lambda i: (i, 0))
_emb_spec = pl.BlockSpec((_BR, 16), lambda i: (i, 0))
_w128_spec = pl.BlockSpec((128, 128), lambda i: (0, 0))
_w16_spec = pl.BlockSpec((16, 128), lambda i: (0, 0))
_b_spec = pl.BlockSpec((1, 128), lambda i: (0, 0))

_fbc_spec = pl.BlockSpec((_BR, 256), lambda i: (i, 0))
_ttab_spec = pl.BlockSpec((256, 16), lambda i: (0, 0))

_tc_in = pl.pallas_call(
    _tc_in_body,
    grid=(_GRID,),
    in_specs=[_row_spec, _row_spec, _row_spec, _fbc_spec, _ttab_spec,
              _w128_spec, _w128_spec, _w16_spec,
              _w128_spec, _w128_spec, _w16_spec],
    out_specs=(_row_spec, _row_spec),
    out_shape=(jax.ShapeDtypeStruct((NROW, 128), jnp.float32),
               jax.ShapeDtypeStruct((NROW, 128), jnp.float32)),
)


def _tc_mid_body(a0_ref, a1_ref, r_ref, b_ref, wr_ref, wo_ref, m_ref, r2_ref):
    h = jnp.maximum(a0_ref[...] + a1_ref[...] + r_ref[...] + b_ref[...], 0.0)
    m_ref[...] = _dot(h, wr_ref[...])
    r2_ref[...] = _dot(h, wo_ref[...])


_tc_mid = pl.pallas_call(
    _tc_mid_body,
    grid=(_GRID,),
    in_specs=[_row_spec, _row_spec, _row_spec, _b_spec, _w128_spec, _w128_spec],
    out_specs=(_row_spec, _row_spec),
    out_shape=(jax.ShapeDtypeStruct((NROW, 128), jnp.float32),
               jax.ShapeDtypeStruct((NROW, 128), jnp.float32)),
)


def _tc_fin_body(a0_ref, a1_ref, r_ref, b_ref, bat_ref,
                 wf1_ref, bf1_ref, wf2_ref, bf2_ref, wf3_ref, bf3_ref,
                 out_ref, pooled):
    i = pl.program_id(0)

    @pl.when(i == 0)
    def _():
        pooled[...] = jnp.full((G, 128), -jnp.inf, jnp.float32)

    h = jnp.maximum(a0_ref[...] + a1_ref[...] + r_ref[...] + b_ref[...], 0.0)
    bat = bat_ref[...]
    gmin = jnp.min(bat)
    gmax = jnp.max(bat)

    def gbody(g, carry):
        sel = jnp.where(bat == g, h, -jnp.inf)
        loc = jnp.max(sel, axis=0, keepdims=True)
        gc = jnp.minimum(g, G - 1)

        @pl.when(g < G)
        def _():
            pooled[pl.ds(gc, 1), :] = jnp.maximum(pooled[pl.ds(gc, 1), :], loc)

        return carry

    lax.fori_loop(gmin, gmax + 1, gbody, 0)

    @pl.when(i == _GRID - 1)
    def _():
        p = pooled[...]
        z = jnp.maximum(_dot(p, wf1_ref[...]) + bf1_ref[...], 0.0)
        z = jnp.maximum(_dot(z, wf2_ref[...]) + bf2_ref[...], 0.0)
        out_ref[...] = _dot(z, wf3_ref[...]) + bf3_ref[...]


_tc_fin = pl.pallas_call(
    _tc_fin_body,
    grid=(_GRID,),
    in_specs=[_row_spec, _row_spec, _row_spec, _b_spec, _row_spec,
              _w128_spec, _b_spec, _w128_spec, _b_spec, _w128_spec, _b_spec],
    out_specs=pl.BlockSpec((G, 128), lambda i: (0, 0)),
    out_shape=jax.ShapeDtypeStruct((G, 128), jnp.float32),
    scratch_shapes=[pltpu.VMEM((G, 128), jnp.float32)],
)


def kernel(x, dst_ports, tcp_flags, edge_index, batch, dst_table, tcp_table,
           Wr0, br0, Wo0, Wr1, br1, Wo1, Wr2, br2, Wo2,
           Wf1, bf1, Wf2, bf2, Wf3, bf3):
    ports = jnp.concatenate([dst_ports.astype(jnp.int32),
                             jnp.arange(NROW - N, dtype=jnp.int32) * 8])
    flags = jnp.pad(tcp_flags.astype(jnp.int32), (0, NROW - N))
    ptab2 = dst_table.reshape(8192, 128)
    epp = _sc_embed()(ports, ptab2)

    src_pad = jnp.arange(_EPAD - E, dtype=jnp.int32) % N
    src2d = jnp.concatenate(
        [edge_index[0].astype(jnp.int32), src_pad]).reshape(_EPAD // _ECH, _ECH)
    dst_pad = N + (jnp.arange(_EPAD - E, dtype=jnp.int32) % (NROW - N))
    dst2d = jnp.concatenate(
        [edge_index[1].astype(jnp.int32), dst_pad]).reshape(_EPAD // _ECH, _ECH)

    zrows = jnp.zeros((_RPT, 128), jnp.float32)
    ports_bc = jnp.broadcast_to(ports[:, None], (NROW, 128))
    flags_bc = jnp.broadcast_to(flags[:, None], (NROW, 256))
    ttab16 = jnp.pad(tcp_table, ((0, 0), (0, 14)))
    m, r = _tc_in(x, epp, ports_bc, flags_bc, ttab16,
                  Wr0[:128], jnp.tile(Wr0[128:144], (8, 1)),
                  jnp.pad(Wr0[144:146], ((0, 14), (0, 0))),
                  Wo0[:128], jnp.tile(Wo0[128:144], (8, 1)),
                  jnp.pad(Wo0[144:146], ((0, 14), (0, 0))))
    a0, a1 = _sc_agg()(m, src2d, dst2d, zrows)
    m, r = _tc_mid(a0, a1, r, br0.reshape(1, 128), Wr1, Wo1)
    a0, a1 = _sc_agg()(m, src2d, dst2d, zrows)
    m, r = _tc_mid(a0, a1, r, br1.reshape(1, 128), Wr2, Wo2)
    a0, a1 = _sc_agg()(m, src2d, dst2d, zrows)

    batp = jnp.broadcast_to(
        jnp.pad(batch.astype(jnp.int32), (0, NROW - N), constant_values=G)[:, None],
        (NROW, 128))
    outf = _tc_fin(a0, a1, r, br2.reshape(1, 128), batp,
                   Wf1, bf1.reshape(1, 128), Wf2, bf2.reshape(1, 128),
                   jnp.pad(Wf3, ((0, 0), (0, 118))),
                   jnp.pad(bf3, (0, 118)).reshape(1, 128))
    return outf[:, :10]


# speedup vs baseline: 2.9217x; 1.0092x over previous
import functools

import jax
import jax.numpy as jnp
from jax import lax
from jax.experimental import pallas as pl
from jax.experimental.pallas import tpu as pltpu
from jax.experimental.pallas import tpu_sc as plsc

N = 10000
E = 320000
NROW = 10240
G = 64
NC, NS = 2, 16
NW = NC * NS

_EMB_CH = 80
_EMB_PER_TILE = NROW // NW


def _sc_embed_body(ports_hbm, ptab2_hbm, epp_hbm, pidx, qidx, prow_v, sem):
    wid = lax.axis_index("s") * NC + lax.axis_index("c")
    base0 = wid * _EMB_PER_TILE
    for k in range(_EMB_PER_TILE // _EMB_CH):
        base = base0 + k * _EMB_CH
        pltpu.sync_copy(ports_hbm.at[pl.ds(base, _EMB_CH)], pidx)
        for g in range(_EMB_CH // 16):
            qidx[pl.ds(16 * g, 16)] = lax.shift_right_logical(
                pidx[pl.ds(16 * g, 16)], 3)
        pltpu.async_copy(ptab2_hbm.at[qidx], prow_v, sem).wait()
        pltpu.sync_copy(prow_v, epp_hbm.at[pl.ds(base, _EMB_CH)])


@functools.cache
def _sc_embed():
    return pl.kernel(
        _sc_embed_body,
        out_type=jax.ShapeDtypeStruct((NROW, 128), jnp.float32),
        mesh=plsc.VectorSubcoreMesh(core_axis_name="c", subcore_axis_name="s",
                                    num_cores=NC, num_subcores=NS),
        scratch_types=[pltpu.VMEM((_EMB_CH,), jnp.int32),
                       pltpu.VMEM((_EMB_CH,), jnp.int32),
                       pltpu.VMEM((_EMB_CH, 128), jnp.float32),
                       pltpu.SemaphoreType.DMA],
    )

_ECH = 128
_EPAD = 327680
_STG = 40
_SLOW_CID = 1
_SLOW_BLK = 2
_FAST_BLK = 2
_RPT = NROW // NS


def _sc_agg_body(m_hbm, src_hbm, dst_hbm, zro_hbm, o0_hbm, o1_hbm,
                 acc, srci, dsti, rows_a, rows_b,
                 zsem, isem, gsem_a, gsem_b, ssem_a, ssem_b):
    cid = lax.axis_index("c")
    sid = lax.axis_index("s")
    rbase = sid * _RPT

    def g_start(c, buf, sem):
        pltpu.async_copy(m_hbm.at[srci.at[c]], buf, sem)

    def g_wait(c, buf, sem):
        pltpu.make_async_copy(m_hbm.at[srci.at[c]], buf, sem).wait()

    def s_start(c, buf, sem):
        pltpu.async_copy(buf, acc.at[dsti.at[c]], sem, add=True)

    def s_wait(c, buf, sem):
        pltpu.make_async_copy(buf, acc.at[dsti.at[c]], sem).wait()

    def stage_idx(trow):
        pltpu.async_copy(src_hbm.at[pl.ds(trow, _STG)], srci, isem)
        pltpu.async_copy(dst_hbm.at[pl.ds(trow, _STG)], dsti, isem)
        pltpu.make_async_copy(src_hbm.at[pl.ds(trow, _STG)], srci, isem).wait()
        pltpu.make_async_copy(dst_hbm.at[pl.ds(trow, _STG)], dsti, isem).wait()

    def pair(j, carry):
        a = 2 * j
        b = a + 1
        g_wait(a, rows_a, gsem_a)
        s_start(a, rows_a, ssem_a)
        g_wait(b, rows_b, gsem_b)
        s_start(b, rows_b, ssem_b)
        s_wait(a, rows_a, ssem_a)
        g_start(a + 2, rows_a, gsem_a)
        s_wait(b, rows_b, ssem_b)
        g_start(b + 2, rows_b, gsem_b)
        return carry

    def run_block(trow):
        stage_idx(trow)
        g_start(0, rows_a, gsem_a)
        g_start(1, rows_b, gsem_b)
        lax.fori_loop(0, _STG // 2 - 1, pair, 0)
        a = _STG - 2
        g_wait(a, rows_a, gsem_a)
        s_start(a, rows_a, ssem_a)
        g_wait(a + 1, rows_b, gsem_b)
        s_start(a + 1, rows_b, ssem_b)
        s_wait(a, rows_a, ssem_a)
        s_wait(a + 1, rows_b, ssem_b)

    pltpu.async_copy(zro_hbm, acc.at[pl.ds(rbase, _RPT)], zsem)
    pltpu.make_async_copy(zro_hbm, acc.at[pl.ds(rbase, _RPT)], zsem).wait()
    plsc.subcore_barrier()

    @pl.when(cid == _SLOW_CID)
    def _():
        for st in range(_SLOW_BLK):
            run_block((sid * _SLOW_BLK + st) * _STG)

    @pl.when(cid != _SLOW_CID)
    def _():
        base = NS * _SLOW_BLK * _STG
        for st in range(_FAST_BLK):
            run_block(base + (sid * _FAST_BLK + st) * _STG)

    plsc.subcore_barrier()

    @pl.when(cid == 0)
    def _():
        pltpu.sync_copy(acc.at[pl.ds(rbase, _RPT)], o0_hbm.at[pl.ds(rbase, _RPT)])

    @pl.when(cid == 1)
    def _():
        pltpu.sync_copy(acc.at[pl.ds(rbase, _RPT)], o1_hbm.at[pl.ds(rbase, _RPT)])


@functools.cache
def _sc_agg():
    return pl.kernel(
        _sc_agg_body,
        out_type=(jax.ShapeDtypeStruct((NROW, 128), jnp.float32),
                  jax.ShapeDtypeStruct((NROW, 128), jnp.float32)),
        mesh=plsc.VectorSubcoreMesh(core_axis_name="c", subcore_axis_name="s",
                                    num_cores=NC, num_subcores=NS),
        scratch_types=[pltpu.VMEM_SHARED((NROW, 128), jnp.float32),
                       pltpu.VMEM((_STG, _ECH), jnp.int32),
                       pltpu.VMEM((_STG, _ECH), jnp.int32),
                       pltpu.VMEM((_ECH, 128), jnp.float32),
                       pltpu.VMEM((_ECH, 128), jnp.float32),
                       pltpu.SemaphoreType.DMA,
                       pltpu.SemaphoreType.DMA,
                       pltpu.SemaphoreType.DMA,
                       pltpu.SemaphoreType.DMA,
                       pltpu.SemaphoreType.DMA,
                       pltpu.SemaphoreType.DMA],
    )

_BR = 512
_GRID = NROW // _BR


def _dot(a, b):
    return jnp.dot(a, b, preferred_element_type=jnp.float32)


def _tc_in_body(x_ref, epp_ref, pc_ref, fc_ref, ttab_ref,
                wxr_ref, wpr_ref, wtr_ref, wxo_ref, wpo_ref, wto_ref,
                m_ref, r_ref):
    xa = x_ref[...]
    lane_grp = lax.shift_right_logical(
        lax.broadcasted_iota(jnp.int32, (_BR, 128), 1), 4)
    rem = pc_ref[...] & 7
    pm = jnp.where(lane_grp == rem, epp_ref[...], 0.0)
    lane = lax.broadcasted_iota(jnp.int32, (_BR, 128), 1)
    fl = fc_ref[...]
    oh1 = jnp.where(fl == lane, 1.0, 0.0)
    oh2 = jnp.where(fl == lane + 128, 1.0, 0.0)
    eta = _dot(oh1, ttab_ref[0:128]) + _dot(oh2, ttab_ref[128:256])
    m_ref[...] = (_dot(xa, wxr_ref[...]) + _dot(pm, wpr_ref[...])
                  + _dot(eta, wtr_ref[...]))
    r_ref[...] = (_dot(xa, wxo_ref[...]) + _dot(pm, wpo_ref[...])
                  + _dot(eta, wto_ref[...]))


_row_spec = pl.BlockSpec((_BR, 128), lambda i: (i, 0))
_emb_spec = pl.BlockSpec((_BR, 16), lambda i: (i, 0))
_w128_spec = pl.BlockSpec((128, 128), lambda i: (0, 0))
_w16_spec = pl.BlockSpec((16, 128), lambda i: (0, 0))
_b_spec = pl.BlockSpec((1, 128), lambda i: (0, 0))

_c1_spec = pl.BlockSpec((_BR, 1), lambda i: (i, 0))
_ttab_spec = pl.BlockSpec((256, 16), lambda i: (0, 0))

_tc_in = pl.pallas_call(
    _tc_in_body,
    grid=(_GRID,),
    in_specs=[_row_spec, _row_spec, _c1_spec, _c1_spec, _ttab_spec,
              _w128_spec, _w128_spec, _w16_spec,
              _w128_spec, _w128_spec, _w16_spec],
    out_specs=(_row_spec, _row_spec),
    out_shape=(jax.ShapeDtypeStruct((NROW, 128), jnp.float32),
               jax.ShapeDtypeStruct((NROW, 128), jnp.float32)),
)


def _tc_mid_body(a0_ref, a1_ref, r_ref, b_ref, wr_ref, wo_ref, m_ref, r2_ref):
    h = jnp.maximum(a0_ref[...] + a1_ref[...] + r_ref[...] + b_ref[...], 0.0)
    m_ref[...] = _dot(h, wr_ref[...])
    r2_ref[...] = _dot(h, wo_ref[...])


_tc_mid = pl.pallas_call(
    _tc_mid_body,
    grid=(_GRID,),
    in_specs=[_row_spec, _row_spec, _row_spec, _b_spec, _w128_spec, _w128_spec],
    out_specs=(_row_spec, _row_spec),
    out_shape=(jax.ShapeDtypeStruct((NROW, 128), jnp.float32),
               jax.ShapeDtypeStruct((NROW, 128), jnp.float32)),
)


def _tc_fin_body(a0_ref, a1_ref, r_ref, b_ref, bat_ref,
                 wf1_ref, bf1_ref, wf2_ref, bf2_ref, wf3_ref, bf3_ref,
                 out_ref, pooled):
    i = pl.program_id(0)

    @pl.when(i == 0)
    def _():
        pooled[...] = jnp.full((G, 128), -jnp.inf, jnp.float32)

    h = jnp.maximum(a0_ref[...] + a1_ref[...] + r_ref[...] + b_ref[...], 0.0)
    bat = bat_ref[...]
    gmin = jnp.min(bat)
    gmax = jnp.max(bat)

    def gbody(g, carry):
        sel = jnp.where(bat == g, h, -jnp.inf)
        loc = jnp.max(sel, axis=0, keepdims=True)
        gc = jnp.minimum(g, G - 1)

        @pl.when(g < G)
        def _():
            pooled[pl.ds(gc, 1), :] = jnp.maximum(pooled[pl.ds(gc, 1), :], loc)

        return carry

    lax.fori_loop(gmin, gmax + 1, gbody, 0)

    @pl.when(i == _GRID - 1)
    def _():
        p = pooled[...]
        z = jnp.maximum(_dot(p, wf1_ref[...]) + bf1_ref[...], 0.0)
        z = jnp.maximum(_dot(z, wf2_ref[...]) + bf2_ref[...], 0.0)
        out_ref[...] = _dot(z, wf3_ref[...]) + bf3_ref[...]


_tc_fin = pl.pallas_call(
    _tc_fin_body,
    grid=(_GRID,),
    in_specs=[_row_spec, _row_spec, _row_spec, _b_spec, _c1_spec,
              _w128_spec, _b_spec, _w128_spec, _b_spec, _w128_spec, _b_spec],
    out_specs=pl.BlockSpec((G, 128), lambda i: (0, 0)),
    out_shape=jax.ShapeDtypeStruct((G, 128), jnp.float32),
    scratch_shapes=[pltpu.VMEM((G, 128), jnp.float32)],
)


def kernel(x, dst_ports, tcp_flags, edge_index, batch, dst_table, tcp_table,
           Wr0, br0, Wo0, Wr1, br1, Wo1, Wr2, br2, Wo2,
           Wf1, bf1, Wf2, bf2, Wf3, bf3):
    ports = jnp.concatenate([dst_ports.astype(jnp.int32),
                             jnp.arange(NROW - N, dtype=jnp.int32) * 8])
    flags = jnp.pad(tcp_flags.astype(jnp.int32), (0, NROW - N))
    ptab2 = dst_table.reshape(8192, 128)
    epp = _sc_embed()(ports, ptab2)

    src_pad = jnp.arange(_EPAD - E, dtype=jnp.int32) % N
    src2d = jnp.concatenate(
        [edge_index[0].astype(jnp.int32), src_pad]).reshape(_EPAD // _ECH, _ECH)
    dst_pad = N + (jnp.arange(_EPAD - E, dtype=jnp.int32) % (NROW - N))
    dst2d = jnp.concatenate(
        [edge_index[1].astype(jnp.int32), dst_pad]).reshape(_EPAD // _ECH, _ECH)

    zrows = jnp.zeros((_RPT, 128), jnp.float32)
    ttab16 = jnp.pad(tcp_table, ((0, 0), (0, 14)))
    m, r = _tc_in(x, epp, ports[:, None], flags[:, None], ttab16,
                  Wr0[:128], jnp.tile(Wr0[128:144], (8, 1)),
                  jnp.pad(Wr0[144:146], ((0, 14), (0, 0))),
                  Wo0[:128], jnp.tile(Wo0[128:144], (8, 1)),
                  jnp.pad(Wo0[144:146], ((0, 14), (0, 0))))
    a0, a1 = _sc_agg()(m, src2d, dst2d, zrows)
    m, r = _tc_mid(a0, a1, r, br0.reshape(1, 128), Wr1, Wo1)
    a0, a1 = _sc_agg()(m, src2d, dst2d, zrows)
    m, r = _tc_mid(a0, a1, r, br1.reshape(1, 128), Wr2, Wo2)
    a0, a1 = _sc_agg()(m, src2d, dst2d, zrows)

    batp = jnp.pad(batch.astype(jnp.int32), (0, NROW - N),
                   constant_values=G)[:, None]
    outf = _tc_fin(a0, a1, r, br2.reshape(1, 128), batp,
                   Wf1, bf1.reshape(1, 128), Wf2, bf2.reshape(1, 128),
                   jnp.pad(Wf3, ((0, 0), (0, 118))),
                   jnp.pad(bf3, (0, 118)).reshape(1, 128))
    return outf[:, :10]
